# Initial kernel scaffold; baseline (speedup 1.0000x reference)
import jax
import jax.numpy as jnp
from jax.experimental import pallas as pl


def kernel(drug_x, drug_edge_index, drug_edge_attr, drug_batch, protein_x, protein_edge_index, protein_edge_attr, protein_batch, params):
    raise NotImplementedError("write your pallas kernel here")


# speedup vs baseline: 1.3865x; 1.3865x over previous
import functools

import jax
import jax.numpy as jnp
from jax.experimental import pallas as pl

_B = 128


def _mlp_head_body(xd_ref, xp_ref,
                   dL1w, dL1b, dL2w, dL2b,
                   pL1w, pL1b, pL2w, pL2b,
                   fW1, fb1, fW2, fb2, fW3, fb3,
                   out_ref):
    relu = jax.nn.relu
    x = xd_ref[...]
    x = relu(jnp.dot(x, dL1w[...], preferred_element_type=jnp.float32) + dL1b[...])
    x = relu(jnp.dot(x, dL2w[...], preferred_element_type=jnp.float32) + dL2b[...])
    p = xp_ref[...]
    p = relu(jnp.dot(p, pL1w[...], preferred_element_type=jnp.float32) + pL1b[...])
    p = relu(jnp.dot(p, pL2w[...], preferred_element_type=jnp.float32) + pL2b[...])
    c = jnp.concatenate([x, p], axis=1)
    h = relu(jnp.dot(c, fW1[...], preferred_element_type=jnp.float32) + fb1[...])
    h = relu(jnp.dot(h, fW2[...], preferred_element_type=jnp.float32) + fb2[...])
    o = jnp.dot(h, fW3[...], preferred_element_type=jnp.float32) + fb3[...]
    out_ref[...] = o


def _mlp_head(xd, xp, params):
    args = (xd, xp,
            params['dL1_w'], params['dL1_b'].reshape(1, -1),
            params['dL2_w'], params['dL2_b'].reshape(1, -1),
            params['pL1_w'], params['pL1_b'].reshape(1, -1),
            params['pL2_w'], params['pL2_b'].reshape(1, -1),
            params['fW1'], params['fb1'].reshape(1, -1),
            params['fW2'], params['fb2'].reshape(1, -1),
            params['fW3'], params['fb3'].reshape(1, -1))
    out = pl.pallas_call(
        _mlp_head_body,
        out_shape=jax.ShapeDtypeStruct((_B, 1), jnp.float32),
    )(*args)
    return out[:, 0]


def _gcn_stack(x, edge_index, edge_attr, batch, W1, W2, W3):
    relu = jax.nn.relu
    N = x.shape[0]
    row, col = edge_index[0], edge_index[1]
    w = edge_attr.reshape(-1)
    ones = jnp.ones((row.shape[0],), dtype=x.dtype)
    deg = jnp.ones((N,), dtype=x.dtype).at[col].add(ones)
    dis = deg ** -0.5
    deg_inv = dis * dis
    norm = dis[row] * w * dis[col]

    def agg(h):
        msg = h[row] * norm[:, None]
        out = (h * deg_inv[:, None]).at[col].add(msg)
        return out

    x = relu(agg(x) @ W1)
    x = relu(agg(x) @ W2)
    x = relu(agg(x) @ W3)

    seg = jax.ops.segment_sum(x, batch, num_segments=_B)
    cnt = jax.ops.segment_sum(jnp.ones((N,), dtype=x.dtype), batch, num_segments=_B)
    return seg / jnp.clip(cnt, 1.0, None)[:, None]


def kernel(drug_x, drug_edge_index, drug_edge_attr, drug_batch,
           protein_x, protein_edge_index, protein_edge_attr, protein_batch,
           params):
    xd = _gcn_stack(drug_x, drug_edge_index, drug_edge_attr, drug_batch,
                    params['dW1'], params['dW2'], params['dW3'])
    xp = _gcn_stack(protein_x, protein_edge_index, protein_edge_attr, protein_batch,
                    params['pW1'], params['pW2'], params['pW3'])
    return _mlp_head(xd, xp, params)


# speedup vs baseline: 5.4915x; 3.9608x over previous
import functools

import jax
import jax.numpy as jnp
from jax import lax
from jax.experimental import pallas as pl
from jax.experimental.pallas import tpu as pltpu
from jax.experimental.pallas import tpu_sc as plsc

_B = 128
NC, NS, LN = 2, 16, 16
N = 50000
NPAD = 50176
SEG = NPAD // NS
E = 800000
EB = 128
BLOCKS = 196
EPT = BLOCKS * EB
EPAD = 32 * EPT
FC = 32
NB = 1024
GRID_N = NPAD // NB

_mesh = plsc.VectorSubcoreMesh(core_axis_name="c", subcore_axis_name="s",
                               num_cores=NC, num_subcores=NS)


def _deg_body(col_ref, valid_ref, out_ref, acc, cbuf, vbuf, zbuf, obuf, sem):
    c = lax.axis_index("c")
    s = lax.axis_index("s")
    for i in range(28):
        zbuf[pl.ds(i * 16, 16)] = jnp.zeros((16,), jnp.float32)
    base = s * SEG
    for i in range(7):
        pltpu.sync_copy(zbuf, acc.at[pl.ds(base + i * 448, 448)])
    plsc.subcore_barrier()
    tile_base = (c * NS + s) * EPT

    def blk(b, _):
        eb = tile_base + b * EB
        pltpu.sync_copy(col_ref.at[pl.ds(eb, EB)], cbuf)
        pltpu.sync_copy(valid_ref.at[pl.ds(eb, EB)], vbuf)
        pltpu.sync_copy(vbuf, acc.at[cbuf], add=True)
        return 0

    lax.fori_loop(0, BLOCKS, blk, 0)
    plsc.subcore_barrier()
    pltpu.sync_copy(acc.at[pl.ds(base, SEG)], obuf)
    pltpu.sync_copy(obuf, out_ref.at[pl.ds(c * NPAD + base, SEG)])


def _deg_kernel(col, valid):
    return pl.kernel(
        _deg_body,
        out_type=jax.ShapeDtypeStruct((NC * NPAD,), jnp.float32),
        mesh=_mesh,
        scratch_types=[
            pltpu.VMEM_SHARED((NPAD,), jnp.float32),
            pltpu.VMEM((EB,), jnp.int32),
            pltpu.VMEM((EB,), jnp.float32),
            pltpu.VMEM((448,), jnp.float32),
            pltpu.VMEM((SEG,), jnp.float32),
            pltpu.SemaphoreType.DMA,
        ],
    )(col, valid)


def _norm_body(row_ref, col_ref, w_ref, dis_ref, out_ref,
               rbuf, cbuf, wbuf, dr, dc, nbuf, sem):
    c = lax.axis_index("c")
    s = lax.axis_index("s")
    tile_base = (c * NS + s) * EPT

    def blk(b, _):
        eb = tile_base + b * EB
        pltpu.sync_copy(row_ref.at[pl.ds(eb, EB)], rbuf)
        pltpu.sync_copy(col_ref.at[pl.ds(eb, EB)], cbuf)
        pltpu.sync_copy(w_ref.at[pl.ds(eb, EB)], wbuf)
        pltpu.make_async_copy(dis_ref.at[rbuf], dr, sem).start()
        pltpu.make_async_copy(dis_ref.at[rbuf], dr, sem).wait()
        pltpu.make_async_copy(dis_ref.at[cbuf], dc, sem).start()
        pltpu.make_async_copy(dis_ref.at[cbuf], dc, sem).wait()
        for g in range(8):
            sl = pl.ds(g * 16, 16)
            nbuf[sl] = dr[sl] * dc[sl] * wbuf[sl]
        pltpu.sync_copy(nbuf, out_ref.at[pl.ds(eb, EB)])
        return 0

    lax.fori_loop(0, BLOCKS, blk, 0)


def _norm_kernel(row, col, w, dis):
    return pl.kernel(
        _norm_body,
        out_type=jax.ShapeDtypeStruct((EPAD,), jnp.float32),
        mesh=_mesh,
        scratch_types=[
            pltpu.VMEM((EB,), jnp.int32),
            pltpu.VMEM((EB,), jnp.int32),
            pltpu.VMEM((EB,), jnp.float32),
            pltpu.VMEM((EB,), jnp.float32),
            pltpu.VMEM((EB,), jnp.float32),
            pltpu.VMEM((EB,), jnp.float32),
            pltpu.SemaphoreType.DMA,
        ],
    )(row, col, w, dis)


def _agg_body(nchunks, xc_ref, row_ref, col_ref, norm_ref, out_ref,
              acc, rbuf, rbuf2, cbuf, nbuf, gbuf, zbuf, obuf, sem):
    c = lax.axis_index("c")
    s = lax.axis_index("s")
    for i in range(112):
        zbuf[i, pl.ds(0, 16)] = jnp.zeros((16,), jnp.float32)
        zbuf[i, pl.ds(16, 16)] = jnp.zeros((16,), jnp.float32)
    base = s * SEG
    tile_base = c * (EPAD // 2) + s * EPT

    for k in range(nchunks):
        for i in range(SEG // 112):
            pltpu.sync_copy(zbuf, acc.at[pl.ds(base + i * 112, 112)])
        plsc.subcore_barrier()

        def blk(b, _):
            eb = tile_base + b * EB
            pltpu.sync_copy(row_ref.at[pl.ds(eb, EB)], rbuf)
            pltpu.sync_copy(col_ref.at[pl.ds(eb, EB)], cbuf)
            pltpu.sync_copy(norm_ref.at[pl.ds(eb, EB)], nbuf)
            for g in range(8):
                sl = pl.ds(g * 16, 16)
                rbuf2[sl] = rbuf[sl] + (k * NPAD)
            pltpu.make_async_copy(xc_ref.at[rbuf2], gbuf, sem).start()
            pltpu.make_async_copy(xc_ref.at[rbuf2], gbuf, sem).wait()

            def egrp(g, _):
                nv = nbuf[pl.ds(g * 16, 16)]
                for l in range(16):
                    e = g * 16 + l
                    nval = nv[l]
                    gbuf[e, pl.ds(0, 16)] = gbuf[e, pl.ds(0, 16)] * nval
                    gbuf[e, pl.ds(16, 16)] = gbuf[e, pl.ds(16, 16)] * nval
                return 0

            lax.fori_loop(0, EB // 16, egrp, 0)
            pltpu.sync_copy(gbuf, acc.at[cbuf], add=True)
            return 0

        lax.fori_loop(0, BLOCKS, blk, 0)
        plsc.subcore_barrier()
        out_row = (c * nchunks + k) * NPAD + base
        for q in range(8):
            pltpu.sync_copy(acc.at[pl.ds(base + q * (SEG // 8), SEG // 8)],
                            obuf)
            pltpu.sync_copy(obuf,
                            out_ref.at[pl.ds(out_row + q * (SEG // 8),
                                             SEG // 8)])


def _agg_kernel(xc, row, col, norm, nchunks):
    return pl.kernel(
        functools.partial(_agg_body, nchunks),
        out_type=jax.ShapeDtypeStruct((NC * nchunks * NPAD, FC), jnp.float32),
        mesh=_mesh,
        compiler_params=pltpu.CompilerParams(use_tc_tiling_on_sc=False),
        scratch_types=[
            pltpu.VMEM_SHARED((NPAD, FC), jnp.float32),
            pltpu.VMEM((EB,), jnp.int32),
            pltpu.VMEM((EB,), jnp.int32),
            pltpu.VMEM((EB,), jnp.int32),
            pltpu.VMEM((EB,), jnp.float32),
            pltpu.VMEM((EB, FC), jnp.float32),
            pltpu.VMEM((112, FC), jnp.float32),
            pltpu.VMEM((SEG // 8, FC), jnp.float32),
            pltpu.SemaphoreType.DMA,
        ],
    )(xc, row, col, norm)


def _layer_body(c1, c2, s_ref, x_ref, dinv_ref, w_ref, out_ref):
    dinv = dinv_ref[...]
    acc = None
    for k in range(c1):
        z = s_ref[0, k] + s_ref[1, k] + x_ref[k] * dinv
        t = jnp.dot(z, w_ref[k], preferred_element_type=jnp.float32)
        acc = t if acc is None else acc + t
    y = jnp.maximum(acc, 0.0)
    for k2 in range(c2):
        out_ref[k2] = y[:, k2 * FC:(k2 + 1) * FC]


def _layer_kernel(s4, xc, dinv, wc):
    c1 = xc.shape[0]
    c2 = wc.shape[2] // FC
    return pl.pallas_call(
        functools.partial(_layer_body, c1, c2),
        grid=(GRID_N,),
        in_specs=[
            pl.BlockSpec((NC, c1, NB, FC), lambda i: (0, 0, i, 0)),
            pl.BlockSpec((c1, NB, FC), lambda i: (0, i, 0)),
            pl.BlockSpec((NB, 1), lambda i: (i, 0)),
            pl.BlockSpec((c1, FC, c2 * FC), lambda i: (0, 0, 0)),
        ],
        out_specs=pl.BlockSpec((c2, NB, FC), lambda i: (0, i, 0)),
        out_shape=jax.ShapeDtypeStruct((c2, NPAD, FC), jnp.float32),
    )(s4, xc, dinv, wc)


def _pool_body(c, y_ref, b_ref, seg_ref, cnt_ref):
    i = pl.program_id(0)

    @pl.when(i == 0)
    def _init():
        seg_ref[...] = jnp.zeros_like(seg_ref)
        cnt_ref[...] = jnp.zeros_like(cnt_ref)

    bids = b_ref[...]
    oh = (bids == lax.broadcasted_iota(jnp.int32, (NB, _B), 1)
          ).astype(jnp.float32)
    for k in range(c):
        seg_ref[k] += lax.dot_general(
            oh, y_ref[k], (((0,), (0,)), ((), ())),
            preferred_element_type=jnp.float32)
    cnt_ref[...] += jnp.sum(oh, axis=0)[:, None]


def _pool_kernel(y, batch2d):
    c = y.shape[0]
    return pl.pallas_call(
        functools.partial(_pool_body, c),
        grid=(GRID_N,),
        in_specs=[
            pl.BlockSpec((c, NB, FC), lambda i: (0, i, 0)),
            pl.BlockSpec((NB, 1), lambda i: (i, 0)),
        ],
        out_specs=[
            pl.BlockSpec((c, _B, FC), lambda i: (0, 0, 0)),
            pl.BlockSpec((_B, 1), lambda i: (0, 0)),
        ],
        out_shape=[
            jax.ShapeDtypeStruct((c, _B, FC), jnp.float32),
            jax.ShapeDtypeStruct((_B, 1), jnp.float32),
        ],
    )(y, batch2d)


def _head_body(cd, cp, dseg, dcnt, pseg, pcnt,
               dL1w, dL1b, dL2w, dL2b, pL1w, pL1b, pL2w, pL2b,
               fW1, fb1, fW2, fb2, fW3, fb3, out_ref):
    relu = lambda v: jnp.maximum(v, 0.0)

    def branch(seg, cnt, w1, b1, w2, b2, chunks):
        inv = 1.0 / jnp.maximum(cnt[...], 1.0)
        acc = None
        for k in range(chunks):
            t = jnp.dot(seg[k] * inv, w1[k],
                        preferred_element_type=jnp.float32)
            acc = t if acc is None else acc + t
        h = relu(acc + b1[...])
        return relu(jnp.dot(h, w2[...], preferred_element_type=jnp.float32)
                    + b2[...])

    x = branch(dseg, dcnt, dL1w, dL1b, dL2w, dL2b, cd)
    p = branch(pseg, pcnt, pL1w, pL1b, pL2w, pL2b, cp)
    cvec = jnp.concatenate([x, p], axis=1)
    h = relu(jnp.dot(cvec, fW1[...], preferred_element_type=jnp.float32)
             + fb1[...])
    h = relu(jnp.dot(h, fW2[...], preferred_element_type=jnp.float32)
             + fb2[...])
    out_ref[...] = (jnp.dot(h, fW3[...], preferred_element_type=jnp.float32)
                    + fb3[...])


def _head_kernel(dseg, dcnt, pseg, pcnt, params):
    cd, cp = dseg.shape[0], pseg.shape[0]
    dL1w = _chunk_w(_pad_rows(params['dL1_w'], cd * FC))
    pL1w = _chunk_w(_pad_rows(params['pL1_w'], cp * FC))
    args = (dseg, dcnt, pseg, pcnt,
            dL1w, params['dL1_b'].reshape(1, -1),
            params['dL2_w'], params['dL2_b'].reshape(1, -1),
            pL1w, params['pL1_b'].reshape(1, -1),
            params['pL2_w'], params['pL2_b'].reshape(1, -1),
            params['fW1'], params['fb1'].reshape(1, -1),
            params['fW2'], params['fb2'].reshape(1, -1),
            params['fW3'], params['fb3'].reshape(1, -1))
    out = pl.pallas_call(
        functools.partial(_head_body, cd, cp),
        out_shape=jax.ShapeDtypeStruct((_B, 1), jnp.float32),
    )(*args)
    return out[:, 0]


def _cdiv(a, b):
    return (a + b - 1) // b


def _pad_rows(w, rows):
    return jnp.pad(w, ((0, rows - w.shape[0]), (0, 0)))


def _chunk_w(w):
    rows, cols = w.shape
    return w.reshape(rows // FC, FC, cols)


def _pack_x(x):
    n, f = x.shape
    c = _cdiv(f, FC)
    xp = jnp.pad(x, ((0, NPAD - n), (0, c * FC - f)))
    return xp.reshape(NPAD, c, FC).transpose(1, 0, 2)


def _gcn_stack(x, edge_index, edge_attr, batch, w1, w2, w3):
    row = jnp.pad(edge_index[0], (0, EPAD - E))
    col = jnp.pad(edge_index[1], (0, EPAD - E))
    wv = jnp.pad(edge_attr.reshape(-1), (0, EPAD - E))
    validv = jnp.pad(jnp.ones((E,), jnp.float32), (0, EPAD - E))

    partial_deg = _deg_kernel(col, validv).reshape(NC, NPAD)
    deg = 1.0 + partial_deg[0] + partial_deg[1]
    dis = deg ** -0.5
    dinv = (1.0 / deg)[:, None]
    norm = _norm_kernel(row, col, wv, dis)

    xc = _pack_x(x)
    ws = []
    fin = x.shape[1]
    for w in (w1, w2, w3):
        c1 = _cdiv(fin, FC)
        c2 = _cdiv(w.shape[1], FC)
        wp = jnp.pad(w, ((0, c1 * FC - w.shape[0]), (0, c2 * FC - w.shape[1])))
        ws.append(wp.reshape(c1, FC, c2 * FC))
        fin = w.shape[1]

    for wc in ws:
        c1 = wc.shape[0]
        s_flat = _agg_kernel(xc.reshape(c1 * NPAD, FC), row, col, norm, c1)
        s4 = s_flat.reshape(NC, c1, NPAD, FC)
        xc = _layer_kernel(s4, xc, dinv, wc)

    batch2d = jnp.pad(batch, (0, NPAD - N),
                      constant_values=1 << 20).astype(jnp.int32)[:, None]
    return _pool_kernel(xc, batch2d)


def kernel(drug_x, drug_edge_index, drug_edge_attr, drug_batch,
           protein_x, protein_edge_index, protein_edge_attr, protein_batch,
           params):
    dseg, dcnt = _gcn_stack(drug_x, drug_edge_index, drug_edge_attr,
                            drug_batch, params['dW1'], params['dW2'],
                            params['dW3'])
    pseg, pcnt = _gcn_stack(protein_x, protein_edge_index, protein_edge_attr,
                            protein_batch, params['pW1'], params['pW2'],
                            params['pW3'])
    return _head_kernel(dseg, dcnt, pseg, pcnt, params)
